# Initial kernel scaffold; baseline (speedup 1.0000x reference)
import jax
import jax.numpy as jnp
from jax.experimental import pallas as pl


def kernel(x, pos, params):
    raise NotImplementedError("write your pallas kernel here")


# speedup vs baseline: 4.0606x; 4.0606x over previous
import functools

import jax
import jax.numpy as jnp
from jax.experimental import pallas as pl

F32 = jnp.float32
K = 16
HIGH = jax.lax.Precision.HIGHEST


def _dot(a, b):
    return jax.lax.dot_general(a, b, (((1,), (0,)), ((), ())),
                               precision=HIGH, preferred_element_type=F32)


def _knn_body(src_ref, q_ref, out_ref, *, n):
    st = src_ref[0]
    q = q_ref[0]
    qs = (st[:, 0:1] * q[0:1, :] + st[:, 1:2] * q[1:2, :]
          + st[:, 2:3] * q[2:3, :])
    ss = st[:, 0:1] ** 2 + st[:, 1:2] ** 2 + st[:, 2:3] ** 2
    qq = q[0:1, :] ** 2 + q[1:2, :] ** 2 + q[2:3, :] ** 2
    d2 = (qq + ss) - 2.0 * qs
    iota = jax.lax.broadcasted_iota(jnp.int32, d2.shape, 0)
    rows = []
    for _ in range(K):
        mn = jnp.min(d2, axis=0, keepdims=True)
        am = jnp.min(jnp.where(d2 == mn, iota, n), axis=0, keepdims=True)
        rows.append(am)
        d2 = jnp.where(iota == am, jnp.inf, d2)
    out_ref[0] = jnp.concatenate(rows, axis=0)


def _knn(src_n3, q8, m):
    b, n, _ = src_n3.shape
    mb = 128 if m >= 128 else m
    grid = (b, m // mb)
    return pl.pallas_call(
        functools.partial(_knn_body, n=n),
        grid=grid,
        in_specs=[
            pl.BlockSpec((1, n, 3), lambda i, j: (i, 0, 0)),
            pl.BlockSpec((1, 8, mb), lambda i, j: (i, 0, j)),
        ],
        out_specs=pl.BlockSpec((1, K, mb), lambda i, j: (i, 0, j)),
        out_shape=jax.ShapeDtypeStruct((b, K, m), jnp.int32),
    )(src_n3, q8[:, :, :m])


def _fka_body(src_ref, sup_ref, ids_ref, fc1_ref, fc2a_ref, fc2b_ref,
              fc3a_ref, fc3b_ref, cv_ref, out_ref, *rest, c_pad, pool_c, n):
    src = src_ref[0]
    sup = sup_ref[0]
    ids = ids_ref[0]
    mb = sup.shape[1]
    iota = jax.lax.broadcasted_iota(jnp.int32, (n, mb), 0)
    g = []
    for k in range(K):
        oh = (iota == ids[k:k + 1, :]).astype(F32)
        g.append(_dot(src, oh))
    pts = [g[k][0:8, :] - sup for k in range(K)]
    dist = [jnp.sqrt(jnp.sum(p * p, axis=0, keepdims=True) + 1e-12)
            for p in pts]
    maxi = dist[0]
    for k in range(1, K):
        maxi = jnp.maximum(maxi, dist[k])
    maxi = jnp.where(maxi <= 0.0, 1.0, maxi)
    fc1 = fc1_ref[...]
    mat1 = [jnp.maximum(_dot(fc1, pts[k] / maxi), 0.0) for k in range(K)]
    mp1 = mat1[0]
    for k in range(1, K):
        mp1 = jnp.maximum(mp1, mat1[k])
    fc2a, fc2b = fc2a_ref[...], fc2b_ref[...]
    mp1t = _dot(fc2b, mp1)
    mat2 = [jnp.maximum(_dot(fc2a, mat1[k]) + mp1t, 0.0) for k in range(K)]
    mp2 = mat2[0]
    for k in range(1, K):
        mp2 = jnp.maximum(mp2, mat2[k])
    fc3a, fc3b = fc3a_ref[...], fc3b_ref[...]
    mp2t = _dot(fc3b, mp2)
    mat3 = [jnp.maximum(_dot(fc3a, mat2[k]) + mp2t, 0.0) for k in range(K)]
    acc = None
    for k in range(K):
        xg = g[k][8:8 + c_pad, :]
        a = (xg[:, None, :] * mat3[k][None, :, :]).reshape(c_pad * K, mb)
        acc = a if acc is None else acc + a
    out_ref[0] = _dot(cv_ref[...], acc)
    if pool_c:
        pool = g[0][8 + c_pad:8 + c_pad + pool_c, :]
        for k in range(1, K):
            pool = jnp.maximum(pool, g[k][8 + c_pad:8 + c_pad + pool_c, :])
        rest[0][0] = pool


def _fka(src_cat, sup8, ids, fcw, oc, c_pad, pool_c, m):
    b, r, n = src_cat.shape
    mb = 128 if m >= 128 else m
    grid = (b, m // mb)
    fc1p, fc2a, fc2b, fc3a, fc3b, cv2d = fcw
    wspec = lambda shp: pl.BlockSpec(shp, lambda i, j: (0, 0))
    out_shape = [jax.ShapeDtypeStruct((b, oc, m), F32)]
    out_specs = [pl.BlockSpec((1, oc, mb), lambda i, j: (i, 0, j))]
    if pool_c:
        out_shape.append(jax.ShapeDtypeStruct((b, pool_c, m), F32))
        out_specs.append(pl.BlockSpec((1, pool_c, mb), lambda i, j: (i, 0, j)))
    res = pl.pallas_call(
        functools.partial(_fka_body, c_pad=c_pad, pool_c=pool_c, n=n),
        grid=grid,
        in_specs=[
            pl.BlockSpec((1, r, n), lambda i, j: (i, 0, 0)),
            pl.BlockSpec((1, 8, mb), lambda i, j: (i, 0, j)),
            pl.BlockSpec((1, K, mb), lambda i, j: (i, 0, j)),
            wspec(fc1p.shape), wspec(fc2a.shape), wspec(fc2b.shape),
            wspec(fc3a.shape), wspec(fc3b.shape), wspec(cv2d.shape),
        ],
        out_specs=out_specs,
        out_shape=out_shape,
    )(src_cat, sup8, ids, fc1p, fc2a, fc2b, fc3a, fc3b, cv2d)
    return res if pool_c else (res[0], None)


def _fka_prep(p, c_pad):
    fc1p = jnp.pad(p['fc1'], ((0, 0), (0, 5)))
    fc2a, fc2b = p['fc2'][:, :K], p['fc2'][:, K:]
    fc3a, fc3b = p['fc3'][:, :K], p['fc3'][:, K:]
    cv = p['cv']
    c = cv.shape[1]
    if c != c_pad:
        cv = jnp.pad(cv, ((0, 0), (0, c_pad - c), (0, 0)))
    cv2d = cv.reshape(cv.shape[0], c_pad * K)
    return fc1p, fc2a, fc2b, fc3a, fc3b, cv2d


def _convbn_body(*refs, nb, inv_cnt, relu, has_res, has_conv):
    i = 0
    x_ref = refs[i]; i += 1
    if has_conv:
        w_ref = refs[i]; b_ref = refs[i + 1]; i += 2
    g_ref = refs[i]; be_ref = refs[i + 1]; i += 2
    r_ref = None
    if has_res:
        r_ref = refs[i]; i += 1
    o_ref = refs[i]
    if has_conv:
        w = w_ref[...]
        bias = b_ref[...]
        ys = [_dot(w, x_ref[j]) + bias for j in range(nb)]
    else:
        ys = [x_ref[j] for j in range(nb)]
    s = jnp.sum(ys[0], axis=1, keepdims=True)
    for j in range(1, nb):
        s = s + jnp.sum(ys[j], axis=1, keepdims=True)
    mean = s * inv_cnt
    v = jnp.sum((ys[0] - mean) ** 2, axis=1, keepdims=True)
    for j in range(1, nb):
        v = v + jnp.sum((ys[j] - mean) ** 2, axis=1, keepdims=True)
    den = jnp.sqrt(v * inv_cnt + 1e-5)
    gamma, beta = g_ref[...], be_ref[...]
    for j in range(nb):
        o = gamma * (ys[j] - mean) / den + beta
        if has_res:
            o = o + r_ref[j]
        if relu:
            o = jnp.maximum(o, 0.0)
        o_ref[j] = o


def _convbn(x, w, bias, gamma, beta, relu=True, residual=None):
    b, _, m = x.shape
    h = w.shape[0]
    args = [x, w, bias.reshape(h, 1), gamma.reshape(h, 1), beta.reshape(h, 1)]
    if residual is not None:
        args.append(residual)
    return pl.pallas_call(
        functools.partial(_convbn_body, nb=b, inv_cnt=1.0 / (b * m),
                          relu=relu, has_res=residual is not None,
                          has_conv=True),
        out_shape=jax.ShapeDtypeStruct((b, h, m), F32),
    )(*args)


def _bn(x, gamma, beta, relu=True):
    b, c, m = x.shape
    return pl.pallas_call(
        functools.partial(_convbn_body, nb=b, inv_cnt=1.0 / (b * m),
                          relu=relu, has_res=False, has_conv=False),
        out_shape=jax.ShapeDtypeStruct((b, c, m), F32),
    )(x, gamma.reshape(c, 1), beta.reshape(c, 1))


def _final_body(x_ref, w_ref, b_ref, o_ref, *, nb):
    xm = jnp.concatenate(
        [jnp.mean(x_ref[j], axis=1, keepdims=True) for j in range(nb)], axis=1)
    o_ref[...] = _dot(w_ref[...], xm) + b_ref[...]


def _final(x4, w, bias):
    b = x4.shape[0]
    oc = w.shape[0]
    out = pl.pallas_call(
        functools.partial(_final_body, nb=b),
        out_shape=jax.ShapeDtypeStruct((oc, b), F32),
    )(x4, w, bias.reshape(oc, 1))
    return out.T


def _resblock(p, x, pos8, ids, m_out):
    b, in_c, n = x.shape
    out_c = p['cv2_w'].shape[0]
    h = in_c // 2
    h1 = _convbn(x, p['cv0_w'], p['cv0_b'], p['bn0_g'], p['bn0_b'], relu=True)
    pool_c = 0
    srcs = [pos8[:, :, :n], h1]
    if 'sc_w' in p:
        xs2 = _convbn(x, p['sc_w'], p['sc_b'], p['bnsc_g'], p['bnsc_b'],
                      relu=False)
        srcs.append(xs2)
        pool_c = out_c
    src_cat = jnp.concatenate(srcs, axis=1)
    fcw = _fka_prep(p['fka'], h)
    fka_raw, pool = _fka(src_cat, pos8[:, :, :m_out], ids, fcw,
                         oc=h, c_pad=h, pool_c=pool_c, m=m_out)
    h2 = _bn(fka_raw, p['bn1_g'], p['bn1_b'], relu=True)
    res = pool if pool_c else x
    return _convbn(h2, p['cv2_w'], p['cv2_b'], p['bn2_g'], p['bn2_b'],
                   relu=True, residual=res)


def kernel(x, pos, params):
    b, n, _ = x.shape
    posT = jnp.transpose(pos, (0, 2, 1))
    xT = jnp.transpose(x, (0, 2, 1))
    zeros5 = jnp.zeros((b, 5, n), F32)
    pos8 = jnp.concatenate([posT, zeros5], axis=1)
    x8 = jnp.concatenate([xT, zeros5], axis=1)

    n1, n2, n3, n4 = n // 4, n // 16, n // 64, n // 256
    ids00 = _knn(pos, pos8, n)
    ids01 = _knn(pos, pos8, n1)
    ids11 = _knn(pos[:, :n1], pos8, n1)
    ids12 = _knn(pos[:, :n1], pos8, n2)
    ids22 = _knn(pos[:, :n2], pos8, n2)
    ids23 = _knn(pos[:, :n2], pos8, n3)
    ids33 = _knn(pos[:, :n3], pos8, n3)
    ids34 = _knn(pos[:, :n3], pos8, n4)
    ids44 = _knn(pos[:, :n4], pos8, n4)

    p = params
    src0 = jnp.concatenate([pos8, x8], axis=1)
    fcw0 = _fka_prep(p['cv0'], 8)
    x0_raw, _ = _fka(src0, pos8, ids00, fcw0, oc=p['cv0']['cv'].shape[0],
                     c_pad=8, pool_c=0, m=n)
    x0 = _bn(x0_raw, p['bn0_g'], p['bn0_b'], relu=True)

    x0 = _resblock(p['b01'], x0, pos8, ids00, n)
    x1 = _resblock(p['b10'], x0, pos8, ids01, n1)
    x1 = _resblock(p['b11'], x1, pos8, ids11, n1)
    x2 = _resblock(p['b20'], x1, pos8, ids12, n2)
    x2 = _resblock(p['b21'], x2, pos8, ids22, n2)
    x3 = _resblock(p['b30'], x2, pos8, ids23, n3)
    x3 = _resblock(p['b31'], x3, pos8, ids33, n3)
    x4 = _resblock(p['b40'], x3, pos8, ids34, n4)
    x4 = _resblock(p['b41'], x4, pos8, ids44, n4)

    return _final(x4, p['fcout_w'], p['fcout_b'])


# speedup vs baseline: 6.8141x; 1.6781x over previous
import functools

import jax
import jax.numpy as jnp
from jax.experimental import pallas as pl

F32 = jnp.float32
K = 16
HIGH = jax.lax.Precision.HIGHEST


def _dot(a, b):
    return jax.lax.dot_general(a, b, (((1,), (0,)), ((), ())),
                               precision=HIGH, preferred_element_type=F32)


def _knn_body(src_ref, q_ref, out_ref, *, n):
    st = src_ref[0]
    q = q_ref[0]
    qs = (st[:, 0:1] * q[0:1, :] + st[:, 1:2] * q[1:2, :]
          + st[:, 2:3] * q[2:3, :])
    ss = st[:, 0:1] ** 2 + st[:, 1:2] ** 2 + st[:, 2:3] ** 2
    qq = q[0:1, :] ** 2 + q[1:2, :] ** 2 + q[2:3, :] ** 2
    d2 = (qq + ss) - 2.0 * qs
    iota = jax.lax.broadcasted_iota(jnp.int32, d2.shape, 0)
    rows = []
    for _ in range(K):
        mn = jnp.min(d2, axis=0, keepdims=True)
        am = jnp.min(jnp.where(d2 == mn, iota, n), axis=0, keepdims=True)
        rows.append(am)
        d2 = jnp.where(iota == am, jnp.inf, d2)
    out_ref[0] = jnp.concatenate(rows, axis=0)


def _knn(src_n3, q8, m):
    b, n, _ = src_n3.shape
    mb = 128 if m >= 128 else m
    grid = (b, m // mb)
    return pl.pallas_call(
        functools.partial(_knn_body, n=n),
        grid=grid,
        in_specs=[
            pl.BlockSpec((1, n, 3), lambda i, j: (i, 0, 0)),
            pl.BlockSpec((1, 8, mb), lambda i, j: (i, 0, j)),
        ],
        out_specs=pl.BlockSpec((1, K, mb), lambda i, j: (i, 0, j)),
        out_shape=jax.ShapeDtypeStruct((b, K, m), jnp.int32),
    )(src_n3, q8[:, :, :m])


def _dotb(a, b):
    return jax.lax.dot_general(a, b, (((1,), (0,)), ((), ())),
                               preferred_element_type=F32)


def _fka_body(hi_ref, lo_ref, sup_ref, ids_ref, fc1_ref, fc2a_ref, fc2b_ref,
              fc3a_ref, fc3b_ref, cv_ref, out_ref, *rest, c_pad, pool_c, n):
    hi = hi_ref[0]
    lo = lo_ref[0]
    sup = sup_ref[0]
    ids = ids_ref[0]
    mb = sup.shape[1]
    iota = jax.lax.broadcasted_iota(jnp.int32, (n, mb), 0)
    g = []
    for k in range(K):
        oh = (iota == ids[k:k + 1, :]).astype(jnp.bfloat16)
        g.append(_dotb(hi, oh) + _dotb(lo, oh))
    ptsc = jnp.concatenate([g[k][0:8, :] - sup for k in range(K)], axis=1)
    distc = jnp.sqrt(jnp.sum(ptsc * ptsc, axis=0, keepdims=True) + 1e-12)
    maxi = distc[:, 0:mb]
    for k in range(1, K):
        maxi = jnp.maximum(maxi, distc[:, k * mb:(k + 1) * mb])
    maxi = jnp.where(maxi <= 0.0, 1.0, maxi)
    maxic = jnp.concatenate([maxi] * K, axis=1)
    mat1 = jnp.maximum(_dot(fc1_ref[...], ptsc / maxic), 0.0)
    mp1 = mat1[:, 0:mb]
    for k in range(1, K):
        mp1 = jnp.maximum(mp1, mat1[:, k * mb:(k + 1) * mb])
    mp1t = _dot(fc2b_ref[...], mp1)
    mat2 = jnp.maximum(_dot(fc2a_ref[...], mat1)
                       + jnp.concatenate([mp1t] * K, axis=1), 0.0)
    mp2 = mat2[:, 0:mb]
    for k in range(1, K):
        mp2 = jnp.maximum(mp2, mat2[:, k * mb:(k + 1) * mb])
    mp2t = _dot(fc3b_ref[...], mp2)
    mat3 = jnp.maximum(_dot(fc3a_ref[...], mat2)
                       + jnp.concatenate([mp2t] * K, axis=1), 0.0)
    acc = None
    for k in range(K):
        xg = g[k][8:8 + c_pad, :]
        m3k = mat3[:, k * mb:(k + 1) * mb]
        a = (xg[:, None, :] * m3k[None, :, :]).reshape(c_pad * K, mb)
        acc = a if acc is None else acc + a
    out_ref[0] = _dot(cv_ref[...], acc)
    if pool_c:
        pool = g[0][8 + c_pad:8 + c_pad + pool_c, :]
        for k in range(1, K):
            pool = jnp.maximum(pool, g[k][8 + c_pad:8 + c_pad + pool_c, :])
        rest[0][0] = pool


def _fka(src_cat, sup8, ids, fcw, oc, c_pad, pool_c, m):
    b, r, n = src_cat.shape
    hi = src_cat.astype(jnp.bfloat16)
    lo = (src_cat - hi.astype(F32)).astype(jnp.bfloat16)
    mb = 128 if m >= 128 else m
    grid = (b, m // mb)
    fc1p, fc2a, fc2b, fc3a, fc3b, cv2d = fcw
    wspec = lambda shp: pl.BlockSpec(shp, lambda i, j: (0, 0))
    out_shape = [jax.ShapeDtypeStruct((b, oc, m), F32)]
    out_specs = [pl.BlockSpec((1, oc, mb), lambda i, j: (i, 0, j))]
    if pool_c:
        out_shape.append(jax.ShapeDtypeStruct((b, pool_c, m), F32))
        out_specs.append(pl.BlockSpec((1, pool_c, mb), lambda i, j: (i, 0, j)))
    res = pl.pallas_call(
        functools.partial(_fka_body, c_pad=c_pad, pool_c=pool_c, n=n),
        grid=grid,
        in_specs=[
            pl.BlockSpec((1, r, n), lambda i, j: (i, 0, 0)),
            pl.BlockSpec((1, r, n), lambda i, j: (i, 0, 0)),
            pl.BlockSpec((1, 8, mb), lambda i, j: (i, 0, j)),
            pl.BlockSpec((1, K, mb), lambda i, j: (i, 0, j)),
            wspec(fc1p.shape), wspec(fc2a.shape), wspec(fc2b.shape),
            wspec(fc3a.shape), wspec(fc3b.shape), wspec(cv2d.shape),
        ],
        out_specs=out_specs,
        out_shape=out_shape,
    )(hi, lo, sup8, ids, fc1p, fc2a, fc2b, fc3a, fc3b, cv2d)
    return res if pool_c else (res[0], None)


def _fka_prep(p, c_pad):
    fc1p = jnp.pad(p['fc1'], ((0, 0), (0, 5)))
    fc2a, fc2b = p['fc2'][:, :K], p['fc2'][:, K:]
    fc3a, fc3b = p['fc3'][:, :K], p['fc3'][:, K:]
    cv = p['cv']
    c = cv.shape[1]
    if c != c_pad:
        cv = jnp.pad(cv, ((0, 0), (0, c_pad - c), (0, 0)))
    cv2d = cv.reshape(cv.shape[0], c_pad * K)
    return fc1p, fc2a, fc2b, fc3a, fc3b, cv2d


def _convbn_body(*refs, nb, inv_cnt, relu, has_res, has_conv):
    i = 0
    x_ref = refs[i]; i += 1
    if has_conv:
        w_ref = refs[i]; b_ref = refs[i + 1]; i += 2
    g_ref = refs[i]; be_ref = refs[i + 1]; i += 2
    r_ref = None
    if has_res:
        r_ref = refs[i]; i += 1
    o_ref = refs[i]
    if has_conv:
        w = w_ref[...]
        bias = b_ref[...]
        ys = [_dot(w, x_ref[j]) + bias for j in range(nb)]
    else:
        ys = [x_ref[j] for j in range(nb)]
    s = jnp.sum(ys[0], axis=1, keepdims=True)
    for j in range(1, nb):
        s = s + jnp.sum(ys[j], axis=1, keepdims=True)
    mean = s * inv_cnt
    v = jnp.sum((ys[0] - mean) ** 2, axis=1, keepdims=True)
    for j in range(1, nb):
        v = v + jnp.sum((ys[j] - mean) ** 2, axis=1, keepdims=True)
    den = jnp.sqrt(v * inv_cnt + 1e-5)
    gamma, beta = g_ref[...], be_ref[...]
    for j in range(nb):
        o = gamma * (ys[j] - mean) / den + beta
        if has_res:
            o = o + r_ref[j]
        if relu:
            o = jnp.maximum(o, 0.0)
        o_ref[j] = o


def _convbn(x, w, bias, gamma, beta, relu=True, residual=None):
    b, _, m = x.shape
    h = w.shape[0]
    args = [x, w, bias.reshape(h, 1), gamma.reshape(h, 1), beta.reshape(h, 1)]
    if residual is not None:
        args.append(residual)
    return pl.pallas_call(
        functools.partial(_convbn_body, nb=b, inv_cnt=1.0 / (b * m),
                          relu=relu, has_res=residual is not None,
                          has_conv=True),
        out_shape=jax.ShapeDtypeStruct((b, h, m), F32),
    )(*args)


def _bn(x, gamma, beta, relu=True):
    b, c, m = x.shape
    return pl.pallas_call(
        functools.partial(_convbn_body, nb=b, inv_cnt=1.0 / (b * m),
                          relu=relu, has_res=False, has_conv=False),
        out_shape=jax.ShapeDtypeStruct((b, c, m), F32),
    )(x, gamma.reshape(c, 1), beta.reshape(c, 1))


def _final_body(x_ref, w_ref, b_ref, o_ref, *, nb):
    xm = jnp.concatenate(
        [jnp.mean(x_ref[j], axis=1, keepdims=True) for j in range(nb)], axis=1)
    o_ref[...] = _dot(w_ref[...], xm) + b_ref[...]


def _final(x4, w, bias):
    b = x4.shape[0]
    oc = w.shape[0]
    out = pl.pallas_call(
        functools.partial(_final_body, nb=b),
        out_shape=jax.ShapeDtypeStruct((oc, b), F32),
    )(x4, w, bias.reshape(oc, 1))
    return out.T


def _resblock(p, x, pos8, ids, m_out):
    b, in_c, n = x.shape
    out_c = p['cv2_w'].shape[0]
    h = in_c // 2
    h1 = _convbn(x, p['cv0_w'], p['cv0_b'], p['bn0_g'], p['bn0_b'], relu=True)
    pool_c = 0
    srcs = [pos8[:, :, :n], h1]
    if 'sc_w' in p:
        xs2 = _convbn(x, p['sc_w'], p['sc_b'], p['bnsc_g'], p['bnsc_b'],
                      relu=False)
        srcs.append(xs2)
        pool_c = out_c
    src_cat = jnp.concatenate(srcs, axis=1)
    fcw = _fka_prep(p['fka'], h)
    fka_raw, pool = _fka(src_cat, pos8[:, :, :m_out], ids, fcw,
                         oc=h, c_pad=h, pool_c=pool_c, m=m_out)
    h2 = _bn(fka_raw, p['bn1_g'], p['bn1_b'], relu=True)
    res = pool if pool_c else x
    return _convbn(h2, p['cv2_w'], p['cv2_b'], p['bn2_g'], p['bn2_b'],
                   relu=True, residual=res)


def kernel(x, pos, params):
    b, n, _ = x.shape
    posT = jnp.transpose(pos, (0, 2, 1))
    xT = jnp.transpose(x, (0, 2, 1))
    zeros5 = jnp.zeros((b, 5, n), F32)
    pos8 = jnp.concatenate([posT, zeros5], axis=1)
    x8 = jnp.concatenate([xT, zeros5], axis=1)

    n1, n2, n3, n4 = n // 4, n // 16, n // 64, n // 256
    ids00 = _knn(pos, pos8, n)
    ids01 = _knn(pos, pos8, n1)
    ids11 = _knn(pos[:, :n1], pos8, n1)
    ids12 = _knn(pos[:, :n1], pos8, n2)
    ids22 = _knn(pos[:, :n2], pos8, n2)
    ids23 = _knn(pos[:, :n2], pos8, n3)
    ids33 = _knn(pos[:, :n3], pos8, n3)
    ids34 = _knn(pos[:, :n3], pos8, n4)
    ids44 = _knn(pos[:, :n4], pos8, n4)

    p = params
    src0 = jnp.concatenate([pos8, x8], axis=1)
    fcw0 = _fka_prep(p['cv0'], 8)
    x0_raw, _ = _fka(src0, pos8, ids00, fcw0, oc=p['cv0']['cv'].shape[0],
                     c_pad=8, pool_c=0, m=n)
    x0 = _bn(x0_raw, p['bn0_g'], p['bn0_b'], relu=True)

    x0 = _resblock(p['b01'], x0, pos8, ids00, n)
    x1 = _resblock(p['b10'], x0, pos8, ids01, n1)
    x1 = _resblock(p['b11'], x1, pos8, ids11, n1)
    x2 = _resblock(p['b20'], x1, pos8, ids12, n2)
    x2 = _resblock(p['b21'], x2, pos8, ids22, n2)
    x3 = _resblock(p['b30'], x2, pos8, ids23, n3)
    x3 = _resblock(p['b31'], x3, pos8, ids33, n3)
    x4 = _resblock(p['b40'], x3, pos8, ids34, n4)
    x4 = _resblock(p['b41'], x4, pos8, ids44, n4)

    return _final(x4, p['fcout_w'], p['fcout_b'])
